# baseline (speedup 1.0000x reference)
import functools

import jax
import jax.numpy as jnp
from jax import lax
from jax.experimental import pallas as pl
from jax.experimental.pallas import tpu as pltpu
from jax.experimental.pallas import tpu_sc as plsc


L = 16


@functools.lru_cache(maxsize=None)
def _build_sc_rowgather(nrows, width, m, nc, ns):
    nw = nc * ns
    rg = 8
    csplit = 8
    ntasks = (m // rg) * csplit
    tpw = -(-ntasks // nw)
    qw = width // csplit
    mpad = ((m + L - 1) // L) * L

    mesh = plsc.VectorSubcoreMesh(core_axis_name="c", subcore_axis_name="s")

    nbuf = 5
    dahead = 2

    @functools.partial(
        pl.kernel,
        out_type=jax.ShapeDtypeStruct((m, width), jnp.float32),
        mesh=mesh,
        scratch_types=(
            [pltpu.VMEM((mpad,), jnp.int32)]
            + [pltpu.VMEM((rg, qw), jnp.float32)] * nbuf
            + [pltpu.SemaphoreType.DMA] * (2 * nbuf)
        ),
        compiler_params=pltpu.CompilerParams(
            needs_layout_passes=False,
            skip_device_barrier=True,
            disable_bounds_checks=True,
            disable_semaphore_checks=True,
        ),
    )
    def sc_rowgather(xt, maskf, out_t, maskv, *bufs_sems):
        vbufs = bufs_sems[:nbuf]
        sins = bufs_sems[nbuf:2 * nbuf]
        souts = bufs_sems[2 * nbuf:]
        wid = lax.axis_index("s") * nc + lax.axis_index("c")
        pltpu.sync_copy(maskf, maskv.at[pl.ds(0, m)])

        def active(t):
            return t * nw + wid < ntasks

        def jq(t):
            k = t * nw + wid
            return (k // csplit) * rg, k % csplit

        def start_in(t):
            jbase, q = jq(t)
            col = pl.multiple_of(q * qw, qw)
            b = t % nbuf

            @pl.when(active(t))
            def _():
                for r in range(rg):
                    msplat = plsc.load_gather(
                        maskv,
                        [jnp.full((L,), r, jnp.int32) + jbase])
                    mj = jnp.max(msplat)
                    pltpu.async_copy(
                        xt.at[pl.ds(mj, 1), pl.ds(col, qw)],
                        vbufs[b].at[pl.ds(r, 1)], sins[b])

        def wait_in(t):
            b = t % nbuf

            @pl.when(active(t))
            def _():
                pltpu.make_async_copy(
                    xt.at[pl.ds(0, rg), pl.ds(0, qw)], vbufs[b],
                    sins[b]).wait()

        def start_out(t):
            jbase, q = jq(t)
            col = pl.multiple_of(q * qw, qw)
            b = t % nbuf

            @pl.when(active(t))
            def _():
                pltpu.async_copy(
                    vbufs[b], out_t.at[pl.ds(jbase, rg), pl.ds(col, qw)],
                    souts[b])

        def wait_out(t):
            b = t % nbuf

            @pl.when(active(t))
            def _():
                pltpu.make_async_copy(
                    vbufs[b], out_t.at[pl.ds(0, rg), pl.ds(0, qw)],
                    souts[b]).wait()

        for t in range(dahead):
            start_in(t)
        for t in range(tpw):
            wait_in(t)
            start_out(t)
            nxt = t + dahead
            if nxt < tpw:
                if nxt - nbuf >= 0:
                    wait_out(nxt - nbuf)
                start_in(nxt)
        for t in range(max(0, tpw - nbuf), tpw):
            wait_out(t)

    return sc_rowgather


def kernel(x, mask):
    n, c = x.shape
    (m,) = mask.shape
    info = plsc.get_sparse_core_info()
    fn = _build_sc_rowgather(c, n, m, info.num_cores, info.num_subcores)
    out_t = fn(x.T, mask.astype(jnp.int32))
    return out_t.T


# speedup vs baseline: 1.1593x; 1.1593x over previous
import functools

import jax
import jax.numpy as jnp
from jax import lax
from jax.experimental import pallas as pl
from jax.experimental.pallas import tpu as pltpu
from jax.experimental.pallas import tpu_sc as plsc


L = 16


@functools.lru_cache(maxsize=None)
def _build_sc_rowgather(nrows, width, m, nc, ns):
    nw = nc * ns
    rg = 2
    csplit = 2
    ntasks = (m // rg) * csplit
    tpw = -(-ntasks // nw)
    qw = width // csplit
    mpad = ((m + L - 1) // L) * L

    mesh = plsc.VectorSubcoreMesh(core_axis_name="c", subcore_axis_name="s")

    nbuf = 6
    dahead = 3

    @functools.partial(
        pl.kernel,
        out_type=jax.ShapeDtypeStruct((m, width), jnp.float32),
        mesh=mesh,
        scratch_types=(
            [pltpu.VMEM((mpad,), jnp.int32)]
            + [pltpu.VMEM((rg, qw), jnp.float32)] * nbuf
            + [pltpu.SemaphoreType.DMA] * (2 * nbuf)
        ),
        compiler_params=pltpu.CompilerParams(
            needs_layout_passes=False,
            skip_device_barrier=True,
            disable_bounds_checks=True,
            disable_semaphore_checks=True,
        ),
    )
    def sc_rowgather(xt, maskf, out_t, maskv, *bufs_sems):
        vbufs = bufs_sems[:nbuf]
        sins = bufs_sems[nbuf:2 * nbuf]
        souts = bufs_sems[2 * nbuf:]
        wid = lax.axis_index("s") * nc + lax.axis_index("c")
        pltpu.sync_copy(maskf, maskv.at[pl.ds(0, m)])

        def active(t):
            return t * nw + wid < ntasks

        def jq(t):
            k = t * nw + wid
            return (k // csplit) * rg, k % csplit

        def start_in(t):
            jbase, q = jq(t)
            col = pl.multiple_of(q * qw, qw)
            b = t % nbuf

            @pl.when(active(t))
            def _():
                for r in range(rg):
                    msplat = plsc.load_gather(
                        maskv,
                        [jnp.full((L,), r, jnp.int32) + jbase])
                    mj = jnp.max(msplat)
                    pltpu.async_copy(
                        xt.at[pl.ds(mj, 1), pl.ds(col, qw)],
                        vbufs[b].at[pl.ds(r, 1)], sins[b])

        def wait_in(t):
            b = t % nbuf

            @pl.when(active(t))
            def _():
                pltpu.make_async_copy(
                    xt.at[pl.ds(0, rg), pl.ds(0, qw)], vbufs[b],
                    sins[b]).wait()

        def start_out(t):
            jbase, q = jq(t)
            col = pl.multiple_of(q * qw, qw)
            b = t % nbuf

            @pl.when(active(t))
            def _():
                pltpu.async_copy(
                    vbufs[b], out_t.at[pl.ds(jbase, rg), pl.ds(col, qw)],
                    souts[b])

        def wait_out(t):
            b = t % nbuf

            @pl.when(active(t))
            def _():
                pltpu.make_async_copy(
                    vbufs[b], out_t.at[pl.ds(0, rg), pl.ds(0, qw)],
                    souts[b]).wait()

        for t in range(dahead):
            start_in(t)
        for t in range(tpw):
            wait_in(t)
            start_out(t)
            nxt = t + dahead
            if nxt < tpw:
                if nxt - nbuf >= 0:
                    wait_out(nxt - nbuf)
                start_in(nxt)
        for t in range(max(0, tpw - nbuf), tpw):
            wait_out(t)

    return sc_rowgather


def kernel(x, mask):
    n, c = x.shape
    (m,) = mask.shape
    info = plsc.get_sparse_core_info()
    fn = _build_sc_rowgather(c, n, m, info.num_cores, info.num_subcores)
    out_t = fn(x.T, mask.astype(jnp.int32))
    return out_t.T


# speedup vs baseline: 1.2075x; 1.0415x over previous
import functools

import jax
import jax.numpy as jnp
from jax import lax
from jax.experimental import pallas as pl
from jax.experimental.pallas import tpu as pltpu
from jax.experimental.pallas import tpu_sc as plsc


L = 16


@functools.lru_cache(maxsize=None)
def _build_sc_rowgather(nrows, width, m, nc, ns):
    nw = nc * ns
    qsplit = 2
    ntasks = m * qsplit
    tpw = -(-ntasks // nw)
    qw = width // qsplit
    mpad = ((m + L - 1) // L) * L

    mesh = plsc.VectorSubcoreMesh(core_axis_name="c", subcore_axis_name="s")

    nbuf = tpw
    dahead = tpw

    @functools.partial(
        pl.kernel,
        out_type=jax.ShapeDtypeStruct((m, width), jnp.float32),
        mesh=mesh,
        scratch_types=(
            [pltpu.VMEM((mpad,), jnp.int32)]
            + [pltpu.VMEM((1, qw), jnp.float32)] * nbuf
            + [pltpu.SemaphoreType.DMA] * (2 * nbuf)
        ),
        compiler_params=pltpu.CompilerParams(
            needs_layout_passes=False,
            skip_device_barrier=True,
            disable_bounds_checks=True,
            disable_semaphore_checks=True,
        ),
    )
    def sc_rowgather(xt, maskf, out_t, maskv, *bufs_sems):
        vbufs = bufs_sems[:nbuf]
        sins = bufs_sems[nbuf:2 * nbuf]
        souts = bufs_sems[2 * nbuf:]
        wid = lax.axis_index("s") * nc + lax.axis_index("c")
        pltpu.sync_copy(maskf, maskv.at[pl.ds(0, m)])

        def active(t):
            return t * nw + wid < ntasks

        def jq(t):
            k = t * nw + wid
            return k // qsplit, k % qsplit

        def start_in(t):
            j, q = jq(t)
            msplat = plsc.load_gather(maskv, [jnp.full((L,), 0, jnp.int32) + j])
            mj = jnp.max(msplat)
            col = pl.multiple_of(q * qw, qw)
            b = t % nbuf

            @pl.when(active(t))
            def _():
                pltpu.async_copy(
                    xt.at[pl.ds(mj, 1), pl.ds(col, qw)], vbufs[b], sins[b])

        def wait_in(t):
            b = t % nbuf

            @pl.when(active(t))
            def _():
                pltpu.make_async_copy(
                    xt.at[pl.ds(0, 1), pl.ds(0, qw)], vbufs[b], sins[b]).wait()

        def start_out(t):
            j, q = jq(t)
            col = pl.multiple_of(q * qw, qw)
            b = t % nbuf

            @pl.when(active(t))
            def _():
                pltpu.async_copy(
                    vbufs[b], out_t.at[pl.ds(j, 1), pl.ds(col, qw)], souts[b])

        def wait_out(t):
            b = t % nbuf

            @pl.when(active(t))
            def _():
                pltpu.make_async_copy(
                    vbufs[b], out_t.at[pl.ds(0, 1), pl.ds(0, qw)],
                    souts[b]).wait()

        for t in range(dahead):
            start_in(t)
        for t in range(tpw):
            wait_in(t)
            start_out(t)
            nxt = t + dahead
            if nxt < tpw:
                if nxt - nbuf >= 0:
                    wait_out(nxt - nbuf)
                start_in(nxt)
        for t in range(max(0, tpw - nbuf), tpw):
            wait_out(t)

    return sc_rowgather


def kernel(x, mask):
    n, c = x.shape
    (m,) = mask.shape
    info = plsc.get_sparse_core_info()
    fn = _build_sc_rowgather(c, n, m, info.num_cores, info.num_subcores)
    out_t = fn(x.T, mask.astype(jnp.int32))
    return out_t.T
